# baseline (speedup 1.0000x reference)
import functools

import jax
import jax.numpy as jnp
from jax import lax
from jax.experimental import pallas as pl
from jax.experimental.pallas import tpu as pltpu
from jax.experimental.pallas import tpu_sc as plsc


def _make_sc_kernel(B, L, W, N, NC, NS):
    NW = NC * NS
    CHUNK = 128
    G = CHUNK // W
    total = B * L * W
    per_w = total // NW
    n_chunks = per_w // CHUNK
    out_rows = B * L
    LANES = 16
    inv_w = 1.0 / W

    mesh = plsc.VectorSubcoreMesh(
        core_axis_name="c", subcore_axis_name="s", num_cores=NC, num_subcores=NS
    )

    @functools.partial(
        pl.kernel,
        out_type=jax.ShapeDtypeStruct((out_rows, N), jnp.float32),
        mesh=mesh,
        scratch_types=[
            pltpu.VMEM((n_chunks, CHUNK), jnp.int32),
            pltpu.VMEM((CHUNK, N), jnp.float32),
            pltpu.VMEM((G, N), jnp.float32),
            pltpu.SemaphoreType.DMA,
        ],
        compiler_params=pltpu.CompilerParams(use_tc_tiling_on_sc=False),
    )
    def sc_kernel(idx_hbm, table_hbm, out_hbm, idx_v, rows_v, out_v, sem):
        wid = lax.axis_index("s") * NC + lax.axis_index("c")
        pltpu.sync_copy(idx_hbm.at[pl.ds(wid * n_chunks, n_chunks)], idx_v)
        out_base = wid * (per_w // W)

        def chunk_body(c, carry):
            pltpu.async_copy(table_hbm.at[idx_v.at[c]], rows_v, sem).wait()
            for g in range(G):
                for lc in range(N // LANES):
                    s = lc * LANES
                    acc = rows_v[W * g, pl.ds(s, LANES)]
                    for j in range(1, W):
                        acc = acc + rows_v[W * g + j, pl.ds(s, LANES)]
                    out_v[g, pl.ds(s, LANES)] = acc * inv_w
            pltpu.sync_copy(out_v, out_hbm.at[pl.ds(out_base + c * G, G)])
            return carry

        lax.fori_loop(0, n_chunks, chunk_body, 0)

    return sc_kernel


def kernel(indices, table):
    B, L, W = indices.shape
    V, N = table.shape
    try:
        info = plsc.get_sparse_core_info()
        NC, NS = info.num_cores, info.num_subcores
    except ValueError:
        NC, NS = 2, 16
    idx = indices.reshape(B * L * W // 128, 128).astype(jnp.int32)
    out = _make_sc_kernel(B, L, W, N, NC, NS)(idx, table)
    return out.reshape(B, L, N)


# speedup vs baseline: 1.1419x; 1.1419x over previous
import functools

import jax
import jax.numpy as jnp
from jax import lax
from jax.experimental import pallas as pl
from jax.experimental.pallas import tpu as pltpu
from jax.experimental.pallas import tpu_sc as plsc

_NB = 4


def _make_sc_kernel(B, L, W, N, NC, NS):
    NW = NC * NS
    CHUNK = 128
    G = CHUNK // W
    total = B * L * W
    per_w = total // NW
    n_chunks = per_w // CHUNK
    n_rounds = n_chunks // _NB
    out_rows = B * L
    LANES = 16
    inv_w = 1.0 / W

    mesh = plsc.VectorSubcoreMesh(
        core_axis_name="c", subcore_axis_name="s", num_cores=NC, num_subcores=NS
    )

    @functools.partial(
        pl.kernel,
        out_type=jax.ShapeDtypeStruct((out_rows, N), jnp.float32),
        mesh=mesh,
        scratch_types=[
            pltpu.VMEM((n_chunks, CHUNK), jnp.int32),
            pltpu.VMEM((_NB, CHUNK, N), jnp.float32),
            pltpu.VMEM((_NB, G, N), jnp.float32),
        ]
        + [pltpu.SemaphoreType.DMA] * (2 * _NB),
        compiler_params=pltpu.CompilerParams(use_tc_tiling_on_sc=False),
    )
    def sc_kernel(idx_hbm, table_hbm, out_hbm, idx_v, rows_v, out_v, *sems):
        gsems, ssems = sems[:_NB], sems[_NB:]
        wid = lax.axis_index("s") * NC + lax.axis_index("c")
        pltpu.sync_copy(idx_hbm.at[pl.ds(wid * n_chunks, n_chunks)], idx_v)
        out_base = wid * (per_w // W)

        for b in range(_NB):
            pltpu.async_copy(table_hbm.at[idx_v.at[b]], rows_v.at[b], gsems[b])

        def round_body(r, carry):
            for b in range(_NB):
                c = r * _NB + b
                pltpu.make_async_copy(
                    table_hbm.at[idx_v.at[c]], rows_v.at[b], gsems[b]
                ).wait()

                @pl.when(r > 0)
                def _():
                    pltpu.make_async_copy(
                        out_v.at[b],
                        out_hbm.at[pl.ds(out_base + (c - _NB) * G, G)],
                        ssems[b],
                    ).wait()

                def gbody(g, _):
                    for lc in range(N // LANES):
                        s = lc * LANES
                        acc = rows_v[b, W * g, pl.ds(s, LANES)]
                        for j in range(1, W):
                            acc = acc + rows_v[b, W * g + j, pl.ds(s, LANES)]
                        out_v[b, g, pl.ds(s, LANES)] = acc * inv_w
                    return _

                lax.fori_loop(0, G, gbody, 0)

                pltpu.async_copy(
                    out_v.at[b], out_hbm.at[pl.ds(out_base + c * G, G)], ssems[b]
                )

                @pl.when(c + _NB < n_chunks)
                def _():
                    pltpu.async_copy(
                        table_hbm.at[idx_v.at[c + _NB]], rows_v.at[b], gsems[b]
                    )

            return carry

        lax.fori_loop(0, n_rounds, round_body, 0)

        for b in range(_NB):
            c = n_chunks - _NB + b
            pltpu.make_async_copy(
                out_v.at[b], out_hbm.at[pl.ds(out_base + c * G, G)], ssems[b]
            ).wait()

    return sc_kernel


def kernel(indices, table):
    B, L, W = indices.shape
    V, N = table.shape
    try:
        info = plsc.get_sparse_core_info()
        NC, NS = info.num_cores, info.num_subcores
    except ValueError:
        NC, NS = 2, 16
    idx = indices.reshape(B * L * W // 128, 128).astype(jnp.int32)
    out = _make_sc_kernel(B, L, W, N, NC, NS)(idx, table)
    return out.reshape(B, L, N)
